# baseline (speedup 1.0000x reference)
import functools

import jax
import jax.numpy as jnp
from jax import lax
from jax.experimental import pallas as pl
from jax.experimental.pallas import tpu as pltpu
from jax.experimental.pallas import tpu_sc as plsc

_TOTAL = 32768
_D = 1024
_NSEG = 16
_NC = 2
_NS = 16
_L = 16
_NW = _NC * _NS
_SCROWS = 10240
_TCROWS = _TOTAL - _SCROWS
_TCBLK = 512
_RPW = _SCROWS // _NW
_CH = 16
_NCH = _RPW // _CH
_NBLK = 4
_BW = _D // _NBLK
_BSTR = _BW // _L


def _sc_segment_sum_body(hs, bnd, out, buf0, buf1, buf2, buf3, bnd_v, acc,
                         sem0, sem1, sem2, sem3):
    cid = lax.axis_index("c")
    sid = lax.axis_index("s")
    wid = sid * _NC + cid
    r0 = wid * _RPW

    pltpu.sync_copy(bnd, bnd_v)
    starts_v = bnd_v[pl.ds(0, _L)]
    ends_v = bnd_v[pl.ds(_L, _L)]
    lane = lax.broadcasted_iota(jnp.int32, (_L,), 0)

    zeros = jnp.zeros((_L,), jnp.float32)

    def zero_body(i, carry):
        acc[pl.ds(i * _L, _L)] = zeros
        return carry

    lax.fori_loop(0, _NSEG * _D // _L, zero_body, 0)

    def chunk_src(c):
        row = pl.multiple_of(r0 + c * _CH, _CH)
        return hs.at[pl.ds(row, _CH)]

    bufs = (buf0, buf1, buf2, buf3)
    sems = (sem0, sem1, sem2, sem3)
    for q in range(4):
        pltpu.async_copy(chunk_src(q), bufs[q], sems[q])

    def seg_count(r):
        return jnp.sum((ends_v <= r).astype(jnp.int32))

    def process(bufref, c):
        glob0 = r0 + c * _CH
        sc_first = seg_count(glob0)
        sc_last = seg_count(glob0 + _CH - 1)

        def seg_body(s, carry):
            m = (lane == s).astype(jnp.int32)
            st = jnp.sum(starts_v * m)
            en = jnp.sum(ends_v * m)
            lo = jnp.maximum(glob0, st) - glob0
            hi = jnp.minimum(glob0 + _CH, en) - glob0
            sbase = s * _D
            for sb in range(_NBLK):
                cb = sb * _BW

                def row_body(r, accs):
                    return tuple(
                        accs[j] + bufref[r, pl.ds(cb + j * _L, _L)]
                        for j in range(_BSTR))

                accs = lax.fori_loop(lo, hi, row_body, (zeros,) * _BSTR)
                for j in range(_BSTR):
                    plsc.addupdate(
                        acc.at[pl.ds(sbase + cb + j * _L, _L)], accs[j])
            return carry

        lax.fori_loop(sc_first, sc_last + 1, seg_body, 0)

    def ring_body(p, carry):
        for q in range(4):
            c = 4 * p + q
            pltpu.make_async_copy(chunk_src(c), bufs[q], sems[q]).wait()
            process(bufs[q], c)

            @pl.when(c + 4 < _NCH)
            def _():
                pltpu.async_copy(chunk_src(c + 4), bufs[q], sems[q])

        return carry

    lax.fori_loop(0, _NCH // 4, ring_body, 0)

    pltpu.sync_copy(acc, out.at[pl.multiple_of(wid, 1)])


_sc_segment_sum = functools.partial(
    pl.kernel,
    out_type=jax.ShapeDtypeStruct((_NW, _NSEG * _D), jnp.float32),
    mesh=plsc.VectorSubcoreMesh(
        core_axis_name="c", subcore_axis_name="s", num_cores=_NC,
        num_subcores=_NS),
    compiler_params=pltpu.CompilerParams(needs_layout_passes=False),
    scratch_types=[
        pltpu.VMEM((_CH, _D), jnp.float32),
        pltpu.VMEM((_CH, _D), jnp.float32),
        pltpu.VMEM((_CH, _D), jnp.float32),
        pltpu.VMEM((_CH, _D), jnp.float32),
        pltpu.VMEM((2 * _L,), jnp.int32),
        pltpu.VMEM((_NSEG * _D,), jnp.float32),
        pltpu.SemaphoreType.DMA,
        pltpu.SemaphoreType.DMA,
        pltpu.SemaphoreType.DMA,
        pltpu.SemaphoreType.DMA,
    ],
)(_sc_segment_sum_body)


def _tc_segsum_body(ends_ref, hs_ref, o_ref):
    i = pl.program_id(0)
    rows = _SCROWS + i * _TCBLK + lax.broadcasted_iota(
        jnp.int32, (1, _TCBLK), 1)
    seg = jnp.zeros((1, _TCBLK), jnp.int32)
    for k in range(_NSEG):
        seg = seg + (rows >= ends_ref[0, k]).astype(jnp.int32)
    sidx = lax.broadcasted_iota(jnp.int32, (_NSEG, _TCBLK), 0)
    onehot = (sidx == seg).astype(jnp.float32)
    partial = jnp.dot(onehot, hs_ref[...],
                      preferred_element_type=jnp.float32)

    @pl.when(i == 0)
    def _():
        o_ref[...] = partial

    @pl.when(i > 0)
    def _():
        o_ref[...] += partial


_tc_segment_sum = pl.pallas_call(
    _tc_segsum_body,
    grid=(_TCROWS // _TCBLK,),
    in_specs=[
        pl.BlockSpec(memory_space=pltpu.SMEM),
        pl.BlockSpec((_TCBLK, _D),
                     lambda i: (_SCROWS // _TCBLK + i, 0)),
    ],
    out_specs=pl.BlockSpec((_NSEG, _D), lambda i: (0, 0)),
    out_shape=jax.ShapeDtypeStruct((_NSEG, _D), jnp.float32),
    compiler_params=pltpu.CompilerParams(
        dimension_semantics=("arbitrary",)),
)


def _finalize_body(p_ref, t_ref, lens_ref, o_ref):
    total = jnp.sum(p_ref[...], axis=0).reshape(_NSEG, _D) + t_ref[...]
    pooled = total / lens_ref[...]
    nrm = jnp.sqrt(jnp.sum(pooled * pooled, axis=1, keepdims=True))
    o_ref[...] = pooled / jnp.maximum(nrm, 1e-12)


def kernel(hidden_states, cu_seqlens):
    bounds = jnp.concatenate([cu_seqlens[:-1], cu_seqlens[1:]])
    ends_tc = cu_seqlens[1:].reshape(1, _NSEG)
    partials = _sc_segment_sum(hidden_states, bounds)
    partial_tc = _tc_segment_sum(ends_tc, hidden_states)
    lens = (cu_seqlens[1:] - cu_seqlens[:-1]).astype(jnp.float32)
    out = pl.pallas_call(
        _finalize_body,
        out_shape=jax.ShapeDtypeStruct((_NSEG, _D), jnp.float32),
    )(partials, partial_tc, lens.reshape(_NSEG, 1))
    return out


# speedup vs baseline: 1.0329x; 1.0329x over previous
import functools

import jax
import jax.numpy as jnp
from jax import lax
from jax.experimental import pallas as pl
from jax.experimental.pallas import tpu as pltpu
from jax.experimental.pallas import tpu_sc as plsc

_TOTAL = 32768
_D = 1024
_NSEG = 16
_NC = 2
_NS = 16
_L = 16
_NW = _NC * _NS
_SCROWS = 20480
_TCROWS = _TOTAL - _SCROWS
_TCBLK = 512
_RPW = _SCROWS // _NW
_CH = 16
_NCH = _RPW // _CH
_NBLK = 4
_BW = _D // _NBLK
_BSTR = _BW // _L


def _sc_segment_sum_body(hs, bnd, out, buf0, buf1, buf2, buf3, bnd_v, acc,
                         sem0, sem1, sem2, sem3):
    cid = lax.axis_index("c")
    sid = lax.axis_index("s")
    wid = sid * _NC + cid
    r0 = wid * _RPW

    pltpu.sync_copy(bnd, bnd_v)
    starts_v = bnd_v[pl.ds(0, _L)]
    ends_v = bnd_v[pl.ds(_L, _L)]
    lane = lax.broadcasted_iota(jnp.int32, (_L,), 0)

    zeros = jnp.zeros((_L,), jnp.float32)

    def zero_body(i, carry):
        acc[pl.ds(i * _L, _L)] = zeros
        return carry

    lax.fori_loop(0, _NSEG * _D // _L, zero_body, 0)

    def chunk_src(c):
        row = pl.multiple_of(r0 + c * _CH, _CH)
        return hs.at[pl.ds(row, _CH)]

    bufs = (buf0, buf1, buf2, buf3)
    sems = (sem0, sem1, sem2, sem3)
    for q in range(4):
        pltpu.async_copy(chunk_src(q), bufs[q], sems[q])

    def seg_count(r):
        return jnp.sum((ends_v <= r).astype(jnp.int32))

    def process(bufref, c):
        glob0 = r0 + c * _CH
        sc_first = seg_count(glob0)
        sc_last = seg_count(glob0 + _CH - 1)

        def seg_body(s, carry):
            m = (lane == s).astype(jnp.int32)
            st = jnp.sum(starts_v * m)
            en = jnp.sum(ends_v * m)
            lo = jnp.maximum(glob0, st) - glob0
            hi = jnp.minimum(glob0 + _CH, en) - glob0
            sbase = s * _D
            for sb in range(_NBLK):
                cb = sb * _BW

                def row_body(r, accs):
                    return tuple(
                        accs[j] + bufref[r, pl.ds(cb + j * _L, _L)]
                        for j in range(_BSTR))

                accs = lax.fori_loop(lo, hi, row_body, (zeros,) * _BSTR)
                for j in range(_BSTR):
                    plsc.addupdate(
                        acc.at[pl.ds(sbase + cb + j * _L, _L)], accs[j])
            return carry

        lax.fori_loop(sc_first, sc_last + 1, seg_body, 0)

    def ring_body(p, carry):
        for q in range(4):
            c = 4 * p + q
            pltpu.make_async_copy(chunk_src(c), bufs[q], sems[q]).wait()
            process(bufs[q], c)

            @pl.when(c + 4 < _NCH)
            def _():
                pltpu.async_copy(chunk_src(c + 4), bufs[q], sems[q])

        return carry

    lax.fori_loop(0, _NCH // 4, ring_body, 0)

    pltpu.sync_copy(acc, out.at[pl.multiple_of(wid, 1)])


_sc_segment_sum = functools.partial(
    pl.kernel,
    out_type=jax.ShapeDtypeStruct((_NW, _NSEG * _D), jnp.float32),
    mesh=plsc.VectorSubcoreMesh(
        core_axis_name="c", subcore_axis_name="s", num_cores=_NC,
        num_subcores=_NS),
    compiler_params=pltpu.CompilerParams(needs_layout_passes=False),
    scratch_types=[
        pltpu.VMEM((_CH, _D), jnp.float32),
        pltpu.VMEM((_CH, _D), jnp.float32),
        pltpu.VMEM((_CH, _D), jnp.float32),
        pltpu.VMEM((_CH, _D), jnp.float32),
        pltpu.VMEM((2 * _L,), jnp.int32),
        pltpu.VMEM((_NSEG * _D,), jnp.float32),
        pltpu.SemaphoreType.DMA,
        pltpu.SemaphoreType.DMA,
        pltpu.SemaphoreType.DMA,
        pltpu.SemaphoreType.DMA,
    ],
)(_sc_segment_sum_body)


def _tc_segsum_body(ends_ref, hs_ref, o_ref):
    i = pl.program_id(0)
    rows = _SCROWS + i * _TCBLK + lax.broadcasted_iota(
        jnp.int32, (1, _TCBLK), 1)
    seg = jnp.zeros((1, _TCBLK), jnp.int32)
    for k in range(_NSEG):
        seg = seg + (rows >= ends_ref[0, k]).astype(jnp.int32)
    sidx = lax.broadcasted_iota(jnp.int32, (_NSEG, _TCBLK), 0)
    onehot = (sidx == seg).astype(jnp.float32)
    partial = jnp.dot(onehot, hs_ref[...],
                      preferred_element_type=jnp.float32)

    @pl.when(i == 0)
    def _():
        o_ref[...] = partial

    @pl.when(i > 0)
    def _():
        o_ref[...] += partial


_tc_segment_sum = pl.pallas_call(
    _tc_segsum_body,
    grid=(_TCROWS // _TCBLK,),
    in_specs=[
        pl.BlockSpec(memory_space=pltpu.SMEM),
        pl.BlockSpec((_TCBLK, _D),
                     lambda i: (_SCROWS // _TCBLK + i, 0)),
    ],
    out_specs=pl.BlockSpec((_NSEG, _D), lambda i: (0, 0)),
    out_shape=jax.ShapeDtypeStruct((_NSEG, _D), jnp.float32),
    compiler_params=pltpu.CompilerParams(
        dimension_semantics=("arbitrary",)),
)


def _finalize_body(p_ref, t_ref, lens_ref, o_ref):
    total = jnp.sum(p_ref[...], axis=0).reshape(_NSEG, _D) + t_ref[...]
    pooled = total / lens_ref[...]
    nrm = jnp.sqrt(jnp.sum(pooled * pooled, axis=1, keepdims=True))
    o_ref[...] = pooled / jnp.maximum(nrm, 1e-12)


def kernel(hidden_states, cu_seqlens):
    bounds = jnp.concatenate([cu_seqlens[:-1], cu_seqlens[1:]])
    ends_tc = cu_seqlens[1:].reshape(1, _NSEG)
    partials = _sc_segment_sum(hidden_states, bounds)
    partial_tc = _tc_segment_sum(ends_tc, hidden_states)
    lens = (cu_seqlens[1:] - cu_seqlens[:-1]).astype(jnp.float32)
    out = pl.pallas_call(
        _finalize_body,
        out_shape=jax.ShapeDtypeStruct((_NSEG, _D), jnp.float32),
    )(partials, partial_tc, lens.reshape(_NSEG, 1))
    return out


# speedup vs baseline: 1.0618x; 1.0280x over previous
import functools

import jax
import jax.numpy as jnp
from jax import lax
from jax.experimental import pallas as pl
from jax.experimental.pallas import tpu as pltpu
from jax.experimental.pallas import tpu_sc as plsc

_TOTAL = 32768
_D = 1024
_NSEG = 16
_NC = 2
_NS = 16
_L = 16
_NW = _NC * _NS
_SCROWS = 16384
_TCROWS = _TOTAL - _SCROWS
_TCBLK = 1024
_RPW = _SCROWS // _NW
_CH = 16
_NCH = _RPW // _CH
_NBLK = 4
_BW = _D // _NBLK
_BSTR = _BW // _L


def _sc_segment_sum_body(hs, bnd, out, buf0, buf1, buf2, buf3, bnd_v, acc,
                         sem0, sem1, sem2, sem3):
    cid = lax.axis_index("c")
    sid = lax.axis_index("s")
    wid = sid * _NC + cid
    r0 = wid * _RPW

    pltpu.sync_copy(bnd, bnd_v)
    starts_v = bnd_v[pl.ds(0, _L)]
    ends_v = bnd_v[pl.ds(_L, _L)]
    lane = lax.broadcasted_iota(jnp.int32, (_L,), 0)

    zeros = jnp.zeros((_L,), jnp.float32)

    def zero_body(i, carry):
        acc[pl.ds(i * _L, _L)] = zeros
        return carry

    lax.fori_loop(0, _NSEG * _D // _L, zero_body, 0)

    def chunk_src(c):
        row = pl.multiple_of(r0 + c * _CH, _CH)
        return hs.at[pl.ds(row, _CH)]

    bufs = (buf0, buf1, buf2, buf3)
    sems = (sem0, sem1, sem2, sem3)
    for q in range(4):
        pltpu.async_copy(chunk_src(q), bufs[q], sems[q])

    def seg_count(r):
        return jnp.sum((ends_v <= r).astype(jnp.int32))

    def process(bufref, c):
        glob0 = r0 + c * _CH
        sc_first = seg_count(glob0)
        sc_last = seg_count(glob0 + _CH - 1)

        def seg_body(s, carry):
            m = (lane == s).astype(jnp.int32)
            st = jnp.sum(starts_v * m)
            en = jnp.sum(ends_v * m)
            lo = jnp.maximum(glob0, st) - glob0
            hi = jnp.minimum(glob0 + _CH, en) - glob0
            sbase = s * _D
            for sb in range(_NBLK):
                cb = sb * _BW

                def row_body(r, accs):
                    return tuple(
                        accs[j] + bufref[r, pl.ds(cb + j * _L, _L)]
                        for j in range(_BSTR))

                accs = lax.fori_loop(lo, hi, row_body, (zeros,) * _BSTR)
                for j in range(_BSTR):
                    plsc.addupdate(
                        acc.at[pl.ds(sbase + cb + j * _L, _L)], accs[j])
            return carry

        lax.fori_loop(sc_first, sc_last + 1, seg_body, 0)

    def ring_body(p, carry):
        for q in range(4):
            c = 4 * p + q
            pltpu.make_async_copy(chunk_src(c), bufs[q], sems[q]).wait()
            process(bufs[q], c)

            @pl.when(c + 4 < _NCH)
            def _():
                pltpu.async_copy(chunk_src(c + 4), bufs[q], sems[q])

        return carry

    lax.fori_loop(0, _NCH // 4, ring_body, 0)

    pltpu.sync_copy(acc, out.at[pl.multiple_of(wid, 1)])


_sc_segment_sum = functools.partial(
    pl.kernel,
    out_type=jax.ShapeDtypeStruct((_NW, _NSEG * _D), jnp.float32),
    mesh=plsc.VectorSubcoreMesh(
        core_axis_name="c", subcore_axis_name="s", num_cores=_NC,
        num_subcores=_NS),
    compiler_params=pltpu.CompilerParams(needs_layout_passes=False),
    scratch_types=[
        pltpu.VMEM((_CH, _D), jnp.float32),
        pltpu.VMEM((_CH, _D), jnp.float32),
        pltpu.VMEM((_CH, _D), jnp.float32),
        pltpu.VMEM((_CH, _D), jnp.float32),
        pltpu.VMEM((2 * _L,), jnp.int32),
        pltpu.VMEM((_NSEG * _D,), jnp.float32),
        pltpu.SemaphoreType.DMA,
        pltpu.SemaphoreType.DMA,
        pltpu.SemaphoreType.DMA,
        pltpu.SemaphoreType.DMA,
    ],
)(_sc_segment_sum_body)


def _tc_segsum_body(ends_ref, hs_ref, o_ref):
    i = pl.program_id(0)
    rows = _SCROWS + i * _TCBLK + lax.broadcasted_iota(
        jnp.int32, (1, _TCBLK), 1)
    seg = jnp.zeros((1, _TCBLK), jnp.int32)
    for k in range(_NSEG):
        seg = seg + (rows >= ends_ref[0, k]).astype(jnp.int32)
    sidx = lax.broadcasted_iota(jnp.int32, (_NSEG, _TCBLK), 0)
    onehot = (sidx == seg).astype(jnp.float32)
    partial = jnp.dot(onehot, hs_ref[...],
                      preferred_element_type=jnp.float32)

    @pl.when(i == 0)
    def _():
        o_ref[...] = partial

    @pl.when(i > 0)
    def _():
        o_ref[...] += partial


_tc_segment_sum = pl.pallas_call(
    _tc_segsum_body,
    grid=(_TCROWS // _TCBLK,),
    in_specs=[
        pl.BlockSpec(memory_space=pltpu.SMEM),
        pl.BlockSpec((_TCBLK, _D),
                     lambda i: (_SCROWS // _TCBLK + i, 0)),
    ],
    out_specs=pl.BlockSpec((_NSEG, _D), lambda i: (0, 0)),
    out_shape=jax.ShapeDtypeStruct((_NSEG, _D), jnp.float32),
    compiler_params=pltpu.CompilerParams(
        dimension_semantics=("arbitrary",)),
)


def _finalize_body(p_ref, t_ref, lens_ref, o_ref):
    total = jnp.sum(p_ref[...], axis=0).reshape(_NSEG, _D) + t_ref[...]
    pooled = total / lens_ref[...]
    nrm = jnp.sqrt(jnp.sum(pooled * pooled, axis=1, keepdims=True))
    o_ref[...] = pooled / jnp.maximum(nrm, 1e-12)


def kernel(hidden_states, cu_seqlens):
    bounds = jnp.concatenate([cu_seqlens[:-1], cu_seqlens[1:]])
    ends_tc = cu_seqlens[1:].reshape(1, _NSEG)
    partials = _sc_segment_sum(hidden_states, bounds)
    partial_tc = _tc_segment_sum(ends_tc, hidden_states)
    lens = (cu_seqlens[1:] - cu_seqlens[:-1]).astype(jnp.float32)
    out = pl.pallas_call(
        _finalize_body,
        out_shape=jax.ShapeDtypeStruct((_NSEG, _D), jnp.float32),
    )(partials, partial_tc, lens.reshape(_NSEG, 1))
    return out


# speedup vs baseline: 1.0745x; 1.0120x over previous
import functools

import jax
import jax.numpy as jnp
from jax import lax
from jax.experimental import pallas as pl
from jax.experimental.pallas import tpu as pltpu
from jax.experimental.pallas import tpu_sc as plsc

_TOTAL = 32768
_D = 1024
_NSEG = 16
_NC = 2
_NS = 16
_L = 16
_NW = _NC * _NS
_SCROWS = 16384
_TCROWS = _TOTAL - _SCROWS
_TCBLK = 2048
_RPW = _SCROWS // _NW
_CH = 16
_NCH = _RPW // _CH
_NBLK = 4
_BW = _D // _NBLK
_BSTR = _BW // _L


def _sc_segment_sum_body(hs, bnd, out, buf0, buf1, buf2, buf3, bnd_v, acc,
                         sem0, sem1, sem2, sem3):
    cid = lax.axis_index("c")
    sid = lax.axis_index("s")
    wid = sid * _NC + cid
    r0 = wid * _RPW

    pltpu.sync_copy(bnd, bnd_v)
    starts_v = bnd_v[pl.ds(0, _L)]
    ends_v = bnd_v[pl.ds(_L, _L)]
    lane = lax.broadcasted_iota(jnp.int32, (_L,), 0)

    zeros = jnp.zeros((_L,), jnp.float32)

    def zero_body(i, carry):
        acc[pl.ds(i * _L, _L)] = zeros
        return carry

    lax.fori_loop(0, _NSEG * _D // _L, zero_body, 0)

    def chunk_src(c):
        row = pl.multiple_of(r0 + c * _CH, _CH)
        return hs.at[pl.ds(row, _CH)]

    bufs = (buf0, buf1, buf2, buf3)
    sems = (sem0, sem1, sem2, sem3)
    for q in range(4):
        pltpu.async_copy(chunk_src(q), bufs[q], sems[q])

    def seg_count(r):
        return jnp.sum((ends_v <= r).astype(jnp.int32))

    def process(bufref, c):
        glob0 = r0 + c * _CH
        sc_first = seg_count(glob0)
        sc_last = seg_count(glob0 + _CH - 1)

        def seg_body(s, carry):
            m = (lane == s).astype(jnp.int32)
            st = jnp.sum(starts_v * m)
            en = jnp.sum(ends_v * m)
            lo = jnp.maximum(glob0, st) - glob0
            hi = jnp.minimum(glob0 + _CH, en) - glob0
            sbase = s * _D
            for sb in range(_NBLK):
                cb = sb * _BW

                def row_body(r, accs):
                    return tuple(
                        accs[j] + bufref[r, pl.ds(cb + j * _L, _L)]
                        for j in range(_BSTR))

                accs = lax.fori_loop(lo, hi, row_body, (zeros,) * _BSTR)
                for j in range(_BSTR):
                    plsc.addupdate(
                        acc.at[pl.ds(sbase + cb + j * _L, _L)], accs[j])
            return carry

        lax.fori_loop(sc_first, sc_last + 1, seg_body, 0)

    def ring_body(p, carry):
        for q in range(4):
            c = 4 * p + q
            pltpu.make_async_copy(chunk_src(c), bufs[q], sems[q]).wait()
            process(bufs[q], c)

            @pl.when(c + 4 < _NCH)
            def _():
                pltpu.async_copy(chunk_src(c + 4), bufs[q], sems[q])

        return carry

    lax.fori_loop(0, _NCH // 4, ring_body, 0)

    pltpu.sync_copy(acc, out.at[pl.multiple_of(wid, 1)])


_sc_segment_sum = functools.partial(
    pl.kernel,
    out_type=jax.ShapeDtypeStruct((_NW, _NSEG * _D), jnp.float32),
    mesh=plsc.VectorSubcoreMesh(
        core_axis_name="c", subcore_axis_name="s", num_cores=_NC,
        num_subcores=_NS),
    compiler_params=pltpu.CompilerParams(needs_layout_passes=False),
    scratch_types=[
        pltpu.VMEM((_CH, _D), jnp.float32),
        pltpu.VMEM((_CH, _D), jnp.float32),
        pltpu.VMEM((_CH, _D), jnp.float32),
        pltpu.VMEM((_CH, _D), jnp.float32),
        pltpu.VMEM((2 * _L,), jnp.int32),
        pltpu.VMEM((_NSEG * _D,), jnp.float32),
        pltpu.SemaphoreType.DMA,
        pltpu.SemaphoreType.DMA,
        pltpu.SemaphoreType.DMA,
        pltpu.SemaphoreType.DMA,
    ],
)(_sc_segment_sum_body)


def _tc_segsum_body(ends_ref, hs_ref, o_ref):
    i = pl.program_id(0)
    rows = _SCROWS + i * _TCBLK + lax.broadcasted_iota(
        jnp.int32, (1, _TCBLK), 1)
    seg = jnp.zeros((1, _TCBLK), jnp.int32)
    for k in range(_NSEG):
        seg = seg + (rows >= ends_ref[0, k]).astype(jnp.int32)
    sidx = lax.broadcasted_iota(jnp.int32, (_NSEG, _TCBLK), 0)
    onehot = (sidx == seg).astype(jnp.float32)
    partial = jnp.dot(onehot, hs_ref[...],
                      preferred_element_type=jnp.float32)

    @pl.when(i == 0)
    def _():
        o_ref[...] = partial

    @pl.when(i > 0)
    def _():
        o_ref[...] += partial


_tc_segment_sum = pl.pallas_call(
    _tc_segsum_body,
    grid=(_TCROWS // _TCBLK,),
    in_specs=[
        pl.BlockSpec(memory_space=pltpu.SMEM),
        pl.BlockSpec((_TCBLK, _D),
                     lambda i: (_SCROWS // _TCBLK + i, 0)),
    ],
    out_specs=pl.BlockSpec((_NSEG, _D), lambda i: (0, 0)),
    out_shape=jax.ShapeDtypeStruct((_NSEG, _D), jnp.float32),
    compiler_params=pltpu.CompilerParams(
        dimension_semantics=("arbitrary",)),
)


def _finalize_body(p_ref, t_ref, lens_ref, o_ref):
    total = jnp.sum(p_ref[...], axis=0).reshape(_NSEG, _D) + t_ref[...]
    pooled = total / lens_ref[...]
    nrm = jnp.sqrt(jnp.sum(pooled * pooled, axis=1, keepdims=True))
    o_ref[...] = pooled / jnp.maximum(nrm, 1e-12)


def kernel(hidden_states, cu_seqlens):
    bounds = jnp.concatenate([cu_seqlens[:-1], cu_seqlens[1:]])
    ends_tc = cu_seqlens[1:].reshape(1, _NSEG)
    partials = _sc_segment_sum(hidden_states, bounds)
    partial_tc = _tc_segment_sum(ends_tc, hidden_states)
    lens = (cu_seqlens[1:] - cu_seqlens[:-1]).astype(jnp.float32)
    out = pl.pallas_call(
        _finalize_body,
        out_shape=jax.ShapeDtypeStruct((_NSEG, _D), jnp.float32),
    )(partials, partial_tc, lens.reshape(_NSEG, 1))
    return out


# speedup vs baseline: 1.0950x; 1.0191x over previous
import functools

import jax
import jax.numpy as jnp
from jax import lax
from jax.experimental import pallas as pl
from jax.experimental.pallas import tpu as pltpu
from jax.experimental.pallas import tpu_sc as plsc

_TOTAL = 32768
_D = 1024
_NSEG = 16
_NC = 2
_NS = 16
_L = 16
_NW = _NC * _NS
_SCROWS = 16384
_TCROWS = _TOTAL - _SCROWS
_TCBLK = 4096
_RPW = _SCROWS // _NW
_CH = 16
_NCH = _RPW // _CH
_NBLK = 4
_BW = _D // _NBLK
_BSTR = _BW // _L


def _sc_segment_sum_body(hs, bnd, out, buf0, buf1, buf2, buf3, bnd_v, acc,
                         sem0, sem1, sem2, sem3):
    cid = lax.axis_index("c")
    sid = lax.axis_index("s")
    wid = sid * _NC + cid
    r0 = wid * _RPW

    pltpu.sync_copy(bnd, bnd_v)
    starts_v = bnd_v[pl.ds(0, _L)]
    ends_v = bnd_v[pl.ds(_L, _L)]
    lane = lax.broadcasted_iota(jnp.int32, (_L,), 0)

    zeros = jnp.zeros((_L,), jnp.float32)

    def zero_body(i, carry):
        acc[pl.ds(i * _L, _L)] = zeros
        return carry

    lax.fori_loop(0, _NSEG * _D // _L, zero_body, 0)

    def chunk_src(c):
        row = pl.multiple_of(r0 + c * _CH, _CH)
        return hs.at[pl.ds(row, _CH)]

    bufs = (buf0, buf1, buf2, buf3)
    sems = (sem0, sem1, sem2, sem3)
    for q in range(4):
        pltpu.async_copy(chunk_src(q), bufs[q], sems[q])

    def seg_count(r):
        return jnp.sum((ends_v <= r).astype(jnp.int32))

    def process(bufref, c):
        glob0 = r0 + c * _CH
        sc_first = seg_count(glob0)
        sc_last = seg_count(glob0 + _CH - 1)

        def seg_body(s, carry):
            m = (lane == s).astype(jnp.int32)
            st = jnp.sum(starts_v * m)
            en = jnp.sum(ends_v * m)
            lo = jnp.maximum(glob0, st) - glob0
            hi = jnp.minimum(glob0 + _CH, en) - glob0
            sbase = s * _D
            for sb in range(_NBLK):
                cb = sb * _BW

                def row_body(r, accs):
                    return tuple(
                        accs[j] + bufref[r, pl.ds(cb + j * _L, _L)]
                        for j in range(_BSTR))

                accs = lax.fori_loop(lo, hi, row_body, (zeros,) * _BSTR)
                for j in range(_BSTR):
                    plsc.addupdate(
                        acc.at[pl.ds(sbase + cb + j * _L, _L)], accs[j])
            return carry

        lax.fori_loop(sc_first, sc_last + 1, seg_body, 0)

    def ring_body(p, carry):
        for q in range(4):
            c = 4 * p + q
            pltpu.make_async_copy(chunk_src(c), bufs[q], sems[q]).wait()
            process(bufs[q], c)

            @pl.when(c + 4 < _NCH)
            def _():
                pltpu.async_copy(chunk_src(c + 4), bufs[q], sems[q])

        return carry

    lax.fori_loop(0, _NCH // 4, ring_body, 0)

    pltpu.sync_copy(acc, out.at[pl.multiple_of(wid, 1)])


_sc_segment_sum = functools.partial(
    pl.kernel,
    out_type=jax.ShapeDtypeStruct((_NW, _NSEG * _D), jnp.float32),
    mesh=plsc.VectorSubcoreMesh(
        core_axis_name="c", subcore_axis_name="s", num_cores=_NC,
        num_subcores=_NS),
    compiler_params=pltpu.CompilerParams(needs_layout_passes=False),
    scratch_types=[
        pltpu.VMEM((_CH, _D), jnp.float32),
        pltpu.VMEM((_CH, _D), jnp.float32),
        pltpu.VMEM((_CH, _D), jnp.float32),
        pltpu.VMEM((_CH, _D), jnp.float32),
        pltpu.VMEM((2 * _L,), jnp.int32),
        pltpu.VMEM((_NSEG * _D,), jnp.float32),
        pltpu.SemaphoreType.DMA,
        pltpu.SemaphoreType.DMA,
        pltpu.SemaphoreType.DMA,
        pltpu.SemaphoreType.DMA,
    ],
)(_sc_segment_sum_body)


def _tc_segsum_body(ends_ref, hs_ref, o_ref):
    i = pl.program_id(0)
    rows = _SCROWS + i * _TCBLK + lax.broadcasted_iota(
        jnp.int32, (1, _TCBLK), 1)
    seg = jnp.zeros((1, _TCBLK), jnp.int32)
    for k in range(_NSEG):
        seg = seg + (rows >= ends_ref[0, k]).astype(jnp.int32)
    sidx = lax.broadcasted_iota(jnp.int32, (_NSEG, _TCBLK), 0)
    onehot = (sidx == seg).astype(jnp.float32)
    partial = jnp.dot(onehot, hs_ref[...],
                      preferred_element_type=jnp.float32)

    @pl.when(i == 0)
    def _():
        o_ref[...] = partial

    @pl.when(i > 0)
    def _():
        o_ref[...] += partial


_tc_segment_sum = pl.pallas_call(
    _tc_segsum_body,
    grid=(_TCROWS // _TCBLK,),
    in_specs=[
        pl.BlockSpec(memory_space=pltpu.SMEM),
        pl.BlockSpec((_TCBLK, _D),
                     lambda i: (_SCROWS // _TCBLK + i, 0)),
    ],
    out_specs=pl.BlockSpec((_NSEG, _D), lambda i: (0, 0)),
    out_shape=jax.ShapeDtypeStruct((_NSEG, _D), jnp.float32),
    compiler_params=pltpu.CompilerParams(
        dimension_semantics=("arbitrary",)),
)


def _finalize_body(p_ref, t_ref, lens_ref, o_ref):
    total = jnp.sum(p_ref[...], axis=0).reshape(_NSEG, _D) + t_ref[...]
    pooled = total / lens_ref[...]
    nrm = jnp.sqrt(jnp.sum(pooled * pooled, axis=1, keepdims=True))
    o_ref[...] = pooled / jnp.maximum(nrm, 1e-12)


def kernel(hidden_states, cu_seqlens):
    bounds = jnp.concatenate([cu_seqlens[:-1], cu_seqlens[1:]])
    ends_tc = cu_seqlens[1:].reshape(1, _NSEG)
    partials = _sc_segment_sum(hidden_states, bounds)
    partial_tc = _tc_segment_sum(ends_tc, hidden_states)
    lens = (cu_seqlens[1:] - cu_seqlens[:-1]).astype(jnp.float32)
    out = pl.pallas_call(
        _finalize_body,
        out_shape=jax.ShapeDtypeStruct((_NSEG, _D), jnp.float32),
    )(partials, partial_tc, lens.reshape(_NSEG, 1))
    return out
